# baseline (speedup 1.0000x reference)
import functools

import jax
import jax.numpy as jnp
from jax import lax
from jax.experimental import pallas as pl
from jax.experimental.pallas import tpu as pltpu
from jax.experimental.pallas import tpu_sc as plsc

_TEMP = 0.05
_M = 100000
_F = 128
_C = 1000
_CP = 1024
_B = 1024
_NC = 2
_NS = 16
_NW = _NC * _NS
_CH = 128
_NFULL = _M // _CH
_TAIL = _M - _NFULL * _CH
_TAIL_BASE = _NFULL * _CH
_KMAX = (_NFULL + _NW - 1) // _NW
_RPW = _B // _NW
_SLICE = _CP // _NS
_MPAD = ((_M + _F - 1) // _F) * _F


def _sc_segment_sum(features, labels, labels2d, indexes, zeros_cf, ones_hbm):
    mesh = plsc.VectorSubcoreMesh(
        core_axis_name="c", subcore_axis_name="s",
        num_cores=_NC, num_subcores=_NS)

    @functools.partial(
        pl.kernel,
        out_type=(
            jax.ShapeDtypeStruct((_NC, _CP, _F), jnp.float32),
            jax.ShapeDtypeStruct((_NC, _CP, _F), jnp.float32),
            jax.ShapeDtypeStruct((_B, _F), jnp.int32),
        ),
        mesh=mesh,
        scratch_types=[
            pltpu.VMEM((_CH, _F), jnp.float32),
            pltpu.VMEM((_CH, _F), jnp.float32),
            pltpu.VMEM((_CH,), jnp.int32),
            pltpu.VMEM((_CH,), jnp.int32),
            pltpu.VMEM((_TAIL, _F), jnp.float32),
            pltpu.VMEM((_TAIL,), jnp.int32),
            pltpu.VMEM((_CH, _F), jnp.float32),
            pltpu.VMEM((_TAIL, _F), jnp.float32),
            pltpu.VMEM((_RPW,), jnp.int32),
            pltpu.VMEM((_RPW, _F), jnp.int32),
            pltpu.VMEM_SHARED((_CP, _F), jnp.float32),
            pltpu.VMEM_SHARED((_CP, _F), jnp.float32),
            pltpu.SemaphoreType.DMA,
            pltpu.SemaphoreType.DMA,
            pltpu.SemaphoreType.DMA,
        ],
    )
    def k(features_h, labels_h, labels2d_h, indexes_h, zeros_cf_h, ones_hbm_h,
          cf_out, nums_out, tgt_out,
          fbuf0, fbuf1, lbuf0, lbuf1, fbuf_t, lbuf_t, ones_v, ones_t,
          idxb, gbuf, cf_acc, nums_acc, sem_r0, sem_r1, sem_t):
        cid = lax.axis_index("c")
        sid = lax.axis_index("s")
        w = cid * _NS + sid
        fbufs = (fbuf0, fbuf1)
        lbufs = (lbuf0, lbuf1)
        sem_r = (sem_r0, sem_r1)

        def issue_read(k_slot):
            b = k_slot % 2
            c = w + k_slot * _NW
            c = jnp.where(c < _NFULL, c, 0)
            off = pl.multiple_of(c * _CH, _CH)
            rf = pltpu.async_copy(
                features_h.at[pl.ds(off, _CH)], fbufs[b], sem_r[b])
            rl = pltpu.async_copy(
                labels_h.at[pl.ds(off, _CH)], lbufs[b], sem_r[b])
            return rf, rl

        rd = [None, None]
        rd[0] = issue_read(0)

        tb = w * _RPW
        pltpu.async_copy(indexes_h.at[pl.ds(tb, _RPW)], idxb, sem_t).wait()
        for j in range(_RPW // 16):
            v = idxb[pl.ds(j * 16, 16)]
            rows16 = lax.shift_right_logical(v, 7)
            pltpu.async_copy(
                labels2d_h.at[rows16], gbuf.at[pl.ds(j * 16, 16)],
                sem_t).wait()
        pltpu.sync_copy(gbuf, tgt_out.at[pl.ds(tb, _RPW)])

        pltpu.sync_copy(ones_hbm_h, ones_v)
        pltpu.sync_copy(ones_hbm_h.at[pl.ds(0, _TAIL)], ones_t)
        base = sid * _SLICE
        pltpu.sync_copy(zeros_cf_h, cf_acc.at[pl.ds(base, _SLICE)])
        pltpu.sync_copy(zeros_cf_h, nums_acc.at[pl.ds(base, _SLICE)])
        plsc.subcore_barrier()

        for ks in range(_KMAX - 1):
            b = ks % 2
            rd[1 - b] = issue_read(ks + 1)
            rd[b][0].wait()
            rd[b][1].wait()
            pltpu.sync_copy(fbufs[b], cf_acc.at[lbufs[b]], add=True)
            pltpu.sync_copy(ones_v, nums_acc.at[lbufs[b]], add=True)

        lb = (_KMAX - 1) % 2
        rd[lb][0].wait()
        rd[lb][1].wait()

        @pl.when(w + (_KMAX - 1) * _NW < _NFULL)
        def _():
            pltpu.sync_copy(fbufs[lb], cf_acc.at[lbufs[lb]], add=True)
            pltpu.sync_copy(ones_v, nums_acc.at[lbufs[lb]], add=True)

        @pl.when(w == _NW - 1)
        def _():
            pltpu.sync_copy(features_h.at[pl.ds(_TAIL_BASE, _TAIL)], fbuf_t)
            pltpu.sync_copy(labels_h.at[pl.ds(_TAIL_BASE, _TAIL)], lbuf_t)
            pltpu.sync_copy(fbuf_t, cf_acc.at[lbuf_t], add=True)
            pltpu.sync_copy(ones_t, nums_acc.at[lbuf_t], add=True)

        plsc.subcore_barrier()
        pltpu.sync_copy(cf_acc.at[pl.ds(base, _SLICE)],
                        cf_out.at[cid, pl.ds(base, _SLICE)])
        pltpu.sync_copy(nums_acc.at[pl.ds(base, _SLICE)],
                        nums_out.at[cid, pl.ds(base, _SLICE)])

    return k(features, labels, labels2d, indexes, zeros_cf, ones_hbm)


def _tc_loss_body(res_ref, cf_ref, nums_ref, trows_ref, idx_ref, out_ref):
    r = res_ref[:]
    nrm = jnp.sqrt(jnp.sum(r * r, axis=1, keepdims=True))
    inputs = r / jnp.clip(nrm, 1e-12, None)
    cf = cf_ref[0] + cf_ref[1]
    nums = nums_ref[0, :, 0:1] + nums_ref[1, :, 0:1]
    sim = lax.dot_general(
        cf, inputs, (((1,), (1,)), ((), ())),
        preferred_element_type=jnp.float32,
        precision=lax.Precision.HIGHEST)
    valid_c = lax.broadcasted_iota(jnp.int32, (_CP, 1), 0) < _C
    mask = ((nums > 0) & valid_c).astype(jnp.float32)
    denom = mask * nums + (1.0 - mask)
    vec = sim / _TEMP / denom
    vec = jnp.where(mask > 0, vec, 0.0)
    exps = jnp.exp(vec)
    masked = exps * mask
    sums = jnp.sum(masked, axis=0, keepdims=True) + 1e-6
    msim = masked / sums
    logp = jnp.log(msim + 1e-6)
    col = jnp.bitwise_and(idx_ref[:], _F - 1)
    lane = lax.broadcasted_iota(jnp.int32, (_B, _F), 1)
    tgt = jnp.sum(jnp.where(lane == col, trows_ref[:], 0),
                  axis=1, keepdims=True).astype(jnp.float32)
    eye = (lax.broadcasted_iota(jnp.int32, (_B, _B), 0)
           == lax.broadcasted_iota(jnp.int32, (_B, _B), 1)
           ).astype(jnp.float32)
    tgt_row = lax.dot_general(
        tgt, eye, (((0,), (0,)), ((), ())),
        preferred_element_type=jnp.float32,
        precision=lax.Precision.HIGHEST)
    cidx = lax.broadcasted_iota(jnp.int32, (_CP, _B), 0)
    picked = jnp.where(cidx == tgt_row.astype(jnp.int32), logp, 0.0)
    out_ref[...] = jnp.reshape(-jnp.sum(picked) / _B, (1, 1))


def _tc_loss(results, cf_parts, nums_parts, trows, idx2):
    return pl.pallas_call(
        _tc_loss_body,
        out_shape=jax.ShapeDtypeStruct((1, 1), jnp.float32),
    )(results, cf_parts, nums_parts, trows, idx2)


def kernel(results, features, feature_weights, indexes, labels, cur_epoch):
    del feature_weights, cur_epoch
    labels_i = labels.astype(jnp.int32)
    idx_i = indexes.astype(jnp.int32)
    labels2d = jnp.concatenate(
        [labels_i, jnp.zeros((_MPAD - _M,), jnp.int32)]).reshape(_MPAD // _F, _F)
    zeros_cf = jnp.zeros((_SLICE, _F), jnp.float32)
    ones_hbm = jnp.ones((_CH, _F), jnp.float32)
    cf_parts, nums_parts, trows = _sc_segment_sum(
        features, labels_i, labels2d, idx_i, zeros_cf, ones_hbm)
    out = _tc_loss(results, cf_parts, nums_parts, trows,
                   idx_i.reshape(_B, 1))
    return out[0, 0]
